# baseline (speedup 1.0000x reference)
import functools

import jax
import jax.numpy as jnp
from jax import lax
from jax.experimental import pallas as pl
from jax.experimental.pallas import tpu as pltpu
from jax.experimental.pallas import tpu_sc as plsc

N1 = 32768
D = 512
N2 = 2048
S = 1024

NC = 2
NS = 16
NDS = 4
DSL = D // NDS
NRG = NS // NDS
NGRP = NC * NRG
RPG = N1 // NGRP
CHUNK = 128
NCHUNK = RPG // CHUNK
ZROWS = S // NRG


def _segsum_sc(tensor1, seg_idx, zeros):
    mesh = plsc.VectorSubcoreMesh(
        core_axis_name="c", subcore_axis_name="s", num_cores=NC, num_subcores=NS
    )

    @functools.partial(
        pl.kernel,
        out_type=jax.ShapeDtypeStruct((NC, NDS, S, DSL), jnp.float32),
        mesh=mesh,
        scratch_types=[
            pltpu.VMEM((NCHUNK, CHUNK), jnp.int32),
            pltpu.VMEM((CHUNK, DSL), jnp.float32),
            pltpu.VMEM((CHUNK, DSL), jnp.float32),
            pltpu.VMEM((CHUNK, DSL), jnp.float32),
            pltpu.VMEM((CHUNK, DSL), jnp.float32),
            pltpu.VMEM_SHARED((NDS, S, DSL), jnp.float32),
            pltpu.SemaphoreType.DMA,
            pltpu.SemaphoreType.DMA,
            pltpu.SemaphoreType.DMA,
            pltpu.SemaphoreType.DMA,
            pltpu.SemaphoreType.DMA,
        ],
    )
    def segsum(t1_hbm, idx_hbm, z_hbm, out_hbm, idx_v, rb0, rb1, rb2, rb3,
               acc_sh, gs0, gs1, gs2, gs3, ssem):
        c = lax.axis_index("c")
        s = lax.axis_index("s")
        rg = s // NDS
        dsi = lax.rem(s, NDS)
        d0 = dsi * DSL
        grp = c * NRG + rg
        base = grp * RPG

        plane = acc_sh.at[dsi]
        bufs = (rb0, rb1, rb2, rb3)
        gsems = (gs0, gs1, gs2, gs3)

        def gather(j, p):
            return pltpu.make_async_copy(
                t1_hbm.at[pl.ds(base + j * CHUNK, CHUNK), pl.ds(d0, DSL)],
                bufs[p],
                gsems[p],
            )

        def scatter_desc(j, p):
            return pltpu.make_async_copy(bufs[p], plane.at[idx_v.at[j]], ssem)

        gather(0, 0).start()
        gather(1, 1).start()

        pltpu.sync_copy(z_hbm, acc_sh.at[dsi, pl.ds(rg * ZROWS, ZROWS)])
        pltpu.sync_copy(idx_hbm.at[grp], idx_v)
        plsc.subcore_barrier()

        def body(jj, carry):
            jb = 4 * jj
            for p in range(4):
                j = jb + p
                gather(j, p).wait()
                pltpu.async_copy(bufs[p], plane.at[idx_v.at[j]], ssem, add=True)

                @pl.when(j >= 2)
                def _():
                    scatter_desc(j - 2, (p + 2) % 4).wait()

                @pl.when(j + 2 < NCHUNK)
                def _():
                    gather(j + 2, (p + 2) % 4).start()
            return carry

        lax.fori_loop(0, NCHUNK // 4, body, 0)
        scatter_desc(NCHUNK - 2, 2).wait()
        scatter_desc(NCHUNK - 1, 3).wait()

        plsc.subcore_barrier()
        pltpu.sync_copy(
            acc_sh.at[dsi, pl.ds(rg * ZROWS, ZROWS)],
            out_hbm.at[c, dsi, pl.ds(rg * ZROWS, ZROWS)],
        )

    return segsum(tensor1, seg_idx, zeros)


def _matmul_tc(partials, tensor2):
    def mm(p_ref, t2_ref, o_ref):
        q = p_ref[0] + p_ref[1]
        agg = jnp.concatenate([q[i] for i in range(NDS)], axis=1)
        o_ref[...] = lax.dot_general(
            agg,
            t2_ref[...],
            (((1,), (1,)), ((), ())),
            preferred_element_type=jnp.float32,
        )

    return pl.pallas_call(
        mm,
        out_shape=jax.ShapeDtypeStruct((S, N2), jnp.float32),
    )(partials, tensor2)


def kernel(tensor1, tensor2, segment_ids):
    seg_idx = segment_ids.astype(jnp.int32).reshape(NGRP, NCHUNK, CHUNK)
    zeros = jnp.zeros((ZROWS, DSL), jnp.float32)
    partials = _segsum_sc(tensor1, seg_idx, zeros)
    return _matmul_tc(partials, tensor2)


# speedup vs baseline: 1.0087x; 1.0087x over previous
import functools

import jax
import jax.numpy as jnp
from jax import lax
from jax.experimental import pallas as pl
from jax.experimental.pallas import tpu as pltpu
from jax.experimental.pallas import tpu_sc as plsc

N1 = 32768
D = 512
N2 = 2048
S = 1024

NC = 2
NS = 16
NDS = 4
DSL = D // NDS
NRG = NS // NDS
NGRP = NC * NRG
RPG = N1 // NGRP
CHUNK = 128
NCHUNK = RPG // CHUNK
ZROWS = S // NRG


def _segsum_sc(tensor1, seg_idx, zeros):
    mesh = plsc.VectorSubcoreMesh(
        core_axis_name="c", subcore_axis_name="s", num_cores=NC, num_subcores=NS
    )

    @functools.partial(
        pl.kernel,
        out_type=jax.ShapeDtypeStruct((NC, NDS, S, DSL), jnp.float32),
        mesh=mesh,
        scratch_types=[
            pltpu.VMEM((NCHUNK, CHUNK), jnp.int32),
            pltpu.VMEM((CHUNK, DSL), jnp.float32),
            pltpu.VMEM((CHUNK, DSL), jnp.float32),
            pltpu.VMEM((CHUNK, DSL), jnp.float32),
            pltpu.VMEM((CHUNK, DSL), jnp.float32),
            pltpu.VMEM_SHARED((NDS, S, DSL), jnp.float32),
            pltpu.SemaphoreType.DMA,
            pltpu.SemaphoreType.DMA,
            pltpu.SemaphoreType.DMA,
            pltpu.SemaphoreType.DMA,
            pltpu.SemaphoreType.DMA,
        ],
    )
    def segsum(t1_hbm, idx_hbm, z_hbm, out_hbm, idx_v, rb0, rb1, rb2, rb3,
               acc_sh, gs0, gs1, gs2, gs3, ssem):
        c = lax.axis_index("c")
        s = lax.axis_index("s")
        rg = s // NDS
        dsi = lax.rem(s, NDS)
        d0 = dsi * DSL
        grp = c * NRG + rg
        base = grp * RPG

        pltpu.sync_copy(z_hbm, acc_sh.at[dsi, pl.ds(rg * ZROWS, ZROWS)])
        pltpu.sync_copy(idx_hbm.at[grp], idx_v)
        plsc.subcore_barrier()

        plane = acc_sh.at[dsi]
        bufs = (rb0, rb1, rb2, rb3)
        gsems = (gs0, gs1, gs2, gs3)

        def gather(j, p):
            return pltpu.make_async_copy(
                t1_hbm.at[pl.ds(base + j * CHUNK, CHUNK), pl.ds(d0, DSL)],
                bufs[p],
                gsems[p],
            )

        def scatter_desc(j, p):
            return pltpu.make_async_copy(bufs[p], plane.at[idx_v.at[j]], ssem)

        gather(0, 0).start()
        gather(1, 1).start()

        def body(jj, carry):
            jb = 4 * jj
            for p in range(4):
                j = jb + p
                gather(j, p).wait()
                pltpu.async_copy(bufs[p], plane.at[idx_v.at[j]], ssem, add=True)

                @pl.when(j >= 2)
                def _():
                    scatter_desc(j - 2, (p + 2) % 4).wait()

                @pl.when(j + 2 < NCHUNK)
                def _():
                    gather(j + 2, (p + 2) % 4).start()
            return carry

        lax.fori_loop(0, NCHUNK // 4, body, 0)
        scatter_desc(NCHUNK - 2, 2).wait()
        scatter_desc(NCHUNK - 1, 3).wait()

        plsc.subcore_barrier()
        pltpu.sync_copy(
            acc_sh.at[dsi, pl.ds(rg * ZROWS, ZROWS)],
            out_hbm.at[c, dsi, pl.ds(rg * ZROWS, ZROWS)],
        )

    return segsum(tensor1, seg_idx, zeros)


def _matmul_tc(partials, tensor2):
    def mm(p_ref, t2_ref, o_ref):
        q = p_ref[0] + p_ref[1]
        agg = jnp.concatenate([q[i] for i in range(NDS)], axis=1)
        o_ref[...] = lax.dot_general(
            agg,
            t2_ref[...],
            (((1,), (1,)), ((), ())),
            preferred_element_type=jnp.float32,
        )

    return pl.pallas_call(
        mm,
        out_shape=jax.ShapeDtypeStruct((S, N2), jnp.float32),
    )(partials, tensor2)


def kernel(tensor1, tensor2, segment_ids):
    seg_idx = segment_ids.astype(jnp.int32).reshape(NGRP, NCHUNK, CHUNK)
    zeros = jnp.zeros((ZROWS, DSL), jnp.float32)
    partials = _segsum_sc(tensor1, seg_idx, zeros)
    return _matmul_tc(partials, tensor2)
